# Initial kernel scaffold; baseline (speedup 1.0000x reference)
import jax
import jax.numpy as jnp
from jax.experimental import pallas as pl


def kernel(feature, edge_index, weight, protein_batch, W1, b1, W2, b2, W3, b3, fcW1, fcb1, fcW2, fcb2):
    raise NotImplementedError("write your pallas kernel here")


# speedup vs baseline: 5.1249x; 5.1249x over previous
import functools

import jax
import jax.numpy as jnp
from jax import lax
from jax.experimental import pallas as pl
from jax.experimental.pallas import tpu as pltpu
from jax.experimental.pallas import tpu_sc as plsc

N = 10000
E = 320000
F = 128
G = 64
NC = 2
NS = 16
B = 80
EPT = E // (NC * NS)
NB = EPT // B
RPT = N // NS
ROW_BLK = 2000
GRID = N // ROW_BLK
NP = 10240
RPD = NP // NS


def _mesh():
    return plsc.VectorSubcoreMesh(
        core_axis_name="c", subcore_axis_name="s", num_cores=NC, num_subcores=NS
    )


def _deg_body(dst3, w3, zdeg, degp, dstv, wv, acc):
    c = lax.axis_index("c")
    s = lax.axis_index("s")
    wid = c * NS + s
    rlo = s * RPD
    pltpu.sync_copy(dst3.at[wid], dstv)
    pltpu.sync_copy(w3.at[wid], wv)
    pltpu.sync_copy(zdeg.at[s], acc.at[pl.ds(rlo, RPD)])
    plsc.subcore_barrier()

    def batch(i, _):
        pltpu.sync_copy(wv.at[i], acc.at[dstv.at[i]], add=True)
        return 0

    lax.fori_loop(0, NB, batch, 0)
    plsc.subcore_barrier()
    pltpu.sync_copy(acc.at[pl.ds(rlo, RPD)], degp.at[c, s])


_deg_kernel = pl.kernel(
    _deg_body,
    out_type=jax.ShapeDtypeStruct((NC, NS, RPD), jnp.float32),
    mesh=_mesh(),
    scratch_types=[
        pltpu.VMEM((NB, B), jnp.int32),
        pltpu.VMEM((NB, B), jnp.float32),
        pltpu.VMEM_SHARED((NP,), jnp.float32),
    ],
)


def _make_spmm(nchunk):
    ni = 4 + nchunk

    def body(*refs):
        src3, dst3, w3 = refs[0], refs[1], refs[2]
        hs = refs[3:3 + nchunk]
        zeros = refs[3 + nchunk]
        outs = refs[ni:ni + nchunk]
        srcb, dstb, wb, rows, acc, sem = refs[ni + nchunk:]

        c = lax.axis_index("c")
        s = lax.axis_index("s")
        wid = c * NS + s

        for k in range(nchunk):
            rlo = s * RPT
            pltpu.sync_copy(zeros.at[s], acc.at[pl.ds(rlo, RPT)])
            plsc.subcore_barrier()

            def batch(i, _):
                pltpu.sync_copy(src3.at[wid, i], srcb)
                pltpu.sync_copy(dst3.at[wid, i], dstb)
                pltpu.sync_copy(w3.at[wid, i], wb)
                pltpu.async_copy(hs[k].at[srcb], rows, sem).wait()
                for jg in range(B // 16):
                    wchunk = wb[pl.ds(jg * 16, 16)]
                    for r2 in range(16):
                        wsp = wchunk.at[jnp.full((16,), r2, jnp.int32)].get(
                            mode="promise_in_bounds")
                        r = jg * 16 + r2
                        for j in range(F // 16):
                            sl = pl.ds(j * 16, 16)
                            rows[r, sl] = rows[r, sl] * wsp
                pltpu.sync_copy(rows, acc.at[dstb], add=True)
                return 0

            lax.fori_loop(0, NB, batch, 0)
            plsc.subcore_barrier()
            pltpu.sync_copy(acc.at[pl.ds(rlo, RPT)], outs[k].at[c, s])
            if k + 1 < nchunk:
                plsc.subcore_barrier()

    return pl.kernel(
        body,
        out_type=[jax.ShapeDtypeStruct((NC, NS, RPT, F), jnp.float32)] * nchunk,
        mesh=_mesh(),
        scratch_types=[
            pltpu.VMEM((B,), jnp.int32),
            pltpu.VMEM((B,), jnp.int32),
            pltpu.VMEM((B,), jnp.float32),
            pltpu.VMEM((B, F), jnp.float32),
            pltpu.VMEM_SHARED((N, F), jnp.float32),
            pltpu.SemaphoreType.DMA,
        ],
    )


_spmm1 = _make_spmm(1)
_spmm2 = _make_spmm(2)
_spmm3 = _make_spmm(4)


def _tc_prep_body(d0_ref, d1_ref, feat_ref, w1_ref, dinv_ref, hs1_ref):
    deg = d0_ref[0, 0, :] + d1_ref[0, 0, :] + 1.0
    dinv = jnp.where(deg > 0, lax.rsqrt(deg), 0.0)
    h = jnp.dot(feat_ref[...], w1_ref[...], preferred_element_type=jnp.float32)
    hs1_ref[...] = h * dinv[:, None]
    dinv_ref[...] = dinv[:, None]


_tc_prep = pl.pallas_call(
    _tc_prep_body,
    grid=(GRID,),
    in_specs=[
        pl.BlockSpec((1, 1, ROW_BLK), lambda i: (i, 0, 0)),
        pl.BlockSpec((1, 1, ROW_BLK), lambda i: (i, 0, 0)),
        pl.BlockSpec((ROW_BLK, F), lambda i: (i, 0)),
        pl.BlockSpec((F, F), lambda i: (0, 0)),
    ],
    out_specs=[
        pl.BlockSpec((ROW_BLK, 1), lambda i: (i, 0)),
        pl.BlockSpec((ROW_BLK, F), lambda i: (i, 0)),
    ],
    out_shape=[
        jax.ShapeDtypeStruct((N, 1), jnp.float32),
        jax.ShapeDtypeStruct((N, F), jnp.float32),
    ],
)


def _make_tc_mid(kin, kout):
    def body(*refs):
        s = refs[:kin]
        hs = refs[kin:2 * kin]
        dinv_ref, b_ref, w_ref = refs[2 * kin], refs[2 * kin + 1], refs[2 * kin + 2]
        outs = refs[2 * kin + 3:]
        dinv = dinv_ref[...]
        x = jnp.concatenate(
            [s[k][0] + s[k][1] + hs[k][...] for k in range(kin)], axis=1
        )
        x = jax.nn.relu(x * dinv + b_ref[...])
        h = jnp.dot(x, w_ref[...], preferred_element_type=jnp.float32)
        for k in range(kout):
            outs[k][...] = h[:, k * F:(k + 1) * F] * dinv

    return pl.pallas_call(
        body,
        grid=(GRID,),
        in_specs=[pl.BlockSpec((NC, ROW_BLK, F), lambda i: (0, i, 0))] * kin
        + [pl.BlockSpec((ROW_BLK, F), lambda i: (i, 0))] * kin
        + [
            pl.BlockSpec((ROW_BLK, 1), lambda i: (i, 0)),
            pl.BlockSpec((1, kin * F), lambda i: (0, 0)),
            pl.BlockSpec((kin * F, kout * F), lambda i: (0, 0)),
        ],
        out_specs=[pl.BlockSpec((ROW_BLK, F), lambda i: (i, 0))] * kout,
        out_shape=[jax.ShapeDtypeStruct((N, F), jnp.float32)] * kout,
    )


_tc_mid2 = _make_tc_mid(1, 2)
_tc_mid3 = _make_tc_mid(2, 4)


def _tc_final_body(s0, s1, s2, s3, h0, h1, h2, h3r, dinv_ref, b3_ref, feat_ref,
                   pb_ref, fw1_ref, fb1_ref, fw2_ref, fb2_ref, out_ref, acc):
    i = pl.program_id(0)

    @pl.when(i == 0)
    def _():
        acc[...] = jnp.zeros_like(acc)

    dinv = dinv_ref[...]
    x = jnp.concatenate(
        [sr[0] + sr[1] + hr[...]
         for sr, hr in ((s0, h0), (s1, h1), (s2, h2), (s3, h3r))], axis=1
    )
    h3 = jax.nn.relu(x * dinv + b3_ref[...])
    gx = jnp.concatenate(
        [h3, feat_ref[...], jnp.ones((ROW_BLK, 1), jnp.float32)], axis=1
    )
    pb = pb_ref[0, 0, :]
    onehot = (
        lax.broadcasted_iota(jnp.int32, (G, ROW_BLK), 0) == pb[None, :]
    ).astype(jnp.float32)
    acc[...] += jnp.dot(onehot, gx, preferred_element_type=jnp.float32)

    @pl.when(i == GRID - 1)
    def _():
        sums = acc[:, : 5 * F]
        cnt = acc[:, 5 * F:]
        gc = sums / jnp.clip(cnt, 1.0)
        z = jax.nn.relu(
            jnp.dot(gc, fw1_ref[...], preferred_element_type=jnp.float32)
            + fb1_ref[...]
        )
        out_ref[...] = (
            jnp.dot(z, fw2_ref[...], preferred_element_type=jnp.float32)
            + fb2_ref[...]
        )


_tc_final = pl.pallas_call(
    _tc_final_body,
    grid=(GRID,),
    in_specs=[pl.BlockSpec((NC, ROW_BLK, F), lambda i: (0, i, 0))] * 4
    + [pl.BlockSpec((ROW_BLK, F), lambda i: (i, 0))] * 4
    + [
        pl.BlockSpec((ROW_BLK, 1), lambda i: (i, 0)),
        pl.BlockSpec((1, 4 * F), lambda i: (0, 0)),
        pl.BlockSpec((ROW_BLK, F), lambda i: (i, 0)),
        pl.BlockSpec((1, 1, ROW_BLK), lambda i: (i, 0, 0)),
        pl.BlockSpec((5 * F, 512), lambda i: (0, 0)),
        pl.BlockSpec((1, 512), lambda i: (0, 0)),
        pl.BlockSpec((512, 1), lambda i: (0, 0)),
        pl.BlockSpec((1, 1), lambda i: (0, 0)),
    ],
    out_specs=pl.BlockSpec((G, 1), lambda i: (0, 0)),
    out_shape=jax.ShapeDtypeStruct((G, 1), jnp.float32),
    scratch_shapes=[pltpu.VMEM((G, 5 * F + 1), jnp.float32)],
)


@jax.jit
def kernel(feature, edge_index, weight, protein_batch,
           W1, b1, W2, b2, W3, b3, fcW1, fcb1, fcW2, fcb2):
    src3 = edge_index[0].reshape(NC * NS, NB, B)
    dst3 = edge_index[1].reshape(NC * NS, NB, B)
    w3e = weight.reshape(NC * NS, NB, B)
    zeros = jnp.zeros((NS, RPT, F), jnp.float32)
    zdeg = jnp.zeros((NS, RPD), jnp.float32)
    pb3 = protein_batch.reshape(GRID, 1, ROW_BLK)

    degp = _deg_kernel(dst3, w3e, zdeg)
    d0 = degp[0].reshape(NP)[:N].reshape(GRID, 1, ROW_BLK)
    d1 = degp[1].reshape(NP)[:N].reshape(GRID, 1, ROW_BLK)
    dinv, hs1 = _tc_prep(d0, d1, feature, W1)
    (s1,) = _spmm1(src3, dst3, w3e, hs1, zeros)
    s1 = s1.reshape(NC, N, F)
    hs2 = _tc_mid2(s1, hs1, dinv, b1.reshape(1, F), W2)
    s2 = [o.reshape(NC, N, F) for o in _spmm2(src3, dst3, w3e, *hs2, zeros)]
    hs3 = _tc_mid3(*s2, *hs2, dinv, b2.reshape(1, 2 * F), W3)
    s3 = [o.reshape(NC, N, F) for o in _spmm3(src3, dst3, w3e, *hs3, zeros)]
    out = _tc_final(*s3, *hs3, dinv, b3.reshape(1, 4 * F), feature, pb3,
                    fcW1, fcb1.reshape(1, 512), fcW2, fcb2.reshape(1, 1))
    return out


# speedup vs baseline: 14.2861x; 2.7876x over previous
import functools

import jax
import jax.numpy as jnp
from jax import lax
from jax.experimental import pallas as pl
from jax.experimental.pallas import tpu as pltpu
from jax.experimental.pallas import tpu_sc as plsc

N = 10000
E = 320000
F = 128
G = 64
NC = 2
NS = 16
B = 80
EPT = E // (NC * NS)
NB = EPT // B
RPT = N // NS
ROW_BLK = 2000
GRID = N // ROW_BLK
NP = 10240
RPD = NP // NS


def _mesh():
    return plsc.VectorSubcoreMesh(
        core_axis_name="c", subcore_axis_name="s", num_cores=NC, num_subcores=NS
    )


def _deg_body(dst3, w3, zdeg, degp, dstv, wv, acc):
    c = lax.axis_index("c")
    s = lax.axis_index("s")
    wid = c * NS + s
    rlo = s * RPD
    pltpu.sync_copy(dst3.at[wid], dstv)
    pltpu.sync_copy(w3.at[wid], wv)
    pltpu.sync_copy(zdeg.at[s], acc.at[pl.ds(rlo, RPD)])
    plsc.subcore_barrier()

    def batch(i, _):
        pltpu.sync_copy(wv.at[i], acc.at[dstv.at[i]], add=True)
        return 0

    lax.fori_loop(0, NB, batch, 0)
    plsc.subcore_barrier()
    pltpu.sync_copy(acc.at[pl.ds(rlo, RPD)], degp.at[c, s])


_deg_kernel = pl.kernel(
    _deg_body,
    out_type=jax.ShapeDtypeStruct((NC, NS, RPD), jnp.float32),
    mesh=_mesh(),
    scratch_types=[
        pltpu.VMEM((NB, B), jnp.int32),
        pltpu.VMEM((NB, B), jnp.float32),
        pltpu.VMEM_SHARED((NP,), jnp.float32),
    ],
)


def _make_spmm(nchunk):
    ni = 4 + nchunk

    def body(*refs):
        src3, dst3, w3 = refs[0], refs[1], refs[2]
        hs = refs[3:3 + nchunk]
        zeros = refs[3 + nchunk]
        outs = refs[ni:ni + nchunk]
        sc = refs[ni + nchunk:]
        srcb, dstb, wb, rows, acc = sc[0], sc[1], sc[2], sc[3], sc[4]
        isem = sc[5:8]
        gsem = sc[8:11]
        ssem = sc[11:14]

        c = lax.axis_index("c")
        s = lax.axis_index("s")
        wid = c * NS + s

        def idx_start(i, t):
            pltpu.async_copy(src3.at[wid, i], srcb.at[t], isem[t])
            pltpu.async_copy(dst3.at[wid, i], dstb.at[t], isem[t])
            pltpu.async_copy(w3.at[wid, i], wb.at[t], isem[t])

        def idx_wait(i, t):
            pltpu.make_async_copy(src3.at[wid, i], srcb.at[t], isem[t]).wait()
            pltpu.make_async_copy(dst3.at[wid, i], dstb.at[t], isem[t]).wait()
            pltpu.make_async_copy(w3.at[wid, i], wb.at[t], isem[t]).wait()

        def gather_start(k, t):
            pltpu.async_copy(hs[k].at[srcb.at[t]], rows.at[t], gsem[t])

        def gather_wait(k, t):
            pltpu.make_async_copy(hs[k].at[srcb.at[t]], rows.at[t],
                                  gsem[t]).wait()

        def scat_start(t):
            pltpu.async_copy(rows.at[t], acc.at[dstb.at[t]], ssem[t], add=True)

        def scat_wait(t):
            pltpu.make_async_copy(rows.at[t], acc.at[dstb.at[t]],
                                  ssem[t]).wait()

        def mul(t):
            def mjg(jg, _):
                wchunk = wb[t, pl.ds(jg * 16, 16)]
                for r2 in range(16):
                    wsp = wchunk.at[jnp.full((16,), r2, jnp.int32)].get(
                        mode="promise_in_bounds")
                    r = jg * 16 + r2
                    for j in range(F // 16):
                        sl = pl.ds(j * 16, 16)
                        rows[t, r, sl] = rows[t, r, sl] * wsp
                return 0

            lax.fori_loop(0, B // 16, mjg, 0)

        NG = (NB - 2) // 3

        for k in range(nchunk):
            rlo = s * RPT
            pltpu.sync_copy(zeros.at[s], acc.at[pl.ds(rlo, RPT)])
            plsc.subcore_barrier()

            for t in range(3):
                idx_start(t, t)
            for t in range(3):
                idx_wait(t, t)
                gather_start(k, t)

            def step(g, t):
                i = 3 * g + t
                tp = (t + 2) % 3
                gather_wait(k, t)

                def prefetch_front():
                    scat_wait(tp)
                    idx_start(i + 2, tp)

                def prefetch_back():
                    idx_wait(i + 2, tp)
                    gather_start(k, tp)

                if t == 0:
                    @pl.when(g > 0)
                    def _():
                        prefetch_front()
                else:
                    prefetch_front()
                mul(t)
                scat_start(t)
                if t == 0:
                    @pl.when(g > 0)
                    def _():
                        prefetch_back()
                else:
                    prefetch_back()

            def group(g, _):
                step(g, 0)
                step(g, 1)
                step(g, 2)
                return 0

            lax.fori_loop(0, NG, group, 0)
            gather_wait(k, 0)
            mul(0)
            scat_start(0)
            scat_wait(2)
            gather_wait(k, 1)
            mul(1)
            scat_start(1)
            scat_wait(0)
            scat_wait(1)
            plsc.subcore_barrier()
            pltpu.sync_copy(acc.at[pl.ds(rlo, RPT)], outs[k].at[c, s])
            if k + 1 < nchunk:
                plsc.subcore_barrier()

    return pl.kernel(
        body,
        out_type=[jax.ShapeDtypeStruct((NC, NS, RPT, F), jnp.float32)] * nchunk,
        mesh=_mesh(),
        scratch_types=[
            pltpu.VMEM((3, B), jnp.int32),
            pltpu.VMEM((3, B), jnp.int32),
            pltpu.VMEM((3, B), jnp.float32),
            pltpu.VMEM((3, B, F), jnp.float32),
            pltpu.VMEM_SHARED((N, F), jnp.float32),
        ] + [pltpu.SemaphoreType.DMA] * 9,
    )


_spmm1 = _make_spmm(1)
_spmm2 = _make_spmm(2)
_spmm3 = _make_spmm(4)


def _tc_prep_body(d0_ref, d1_ref, feat_ref, w1_ref, dinv_ref, hs1_ref):
    deg = d0_ref[0, 0, :] + d1_ref[0, 0, :] + 1.0
    dinv = jnp.where(deg > 0, lax.rsqrt(deg), 0.0)
    h = jnp.dot(feat_ref[...], w1_ref[...], preferred_element_type=jnp.float32)
    hs1_ref[...] = h * dinv[:, None]
    dinv_ref[...] = dinv[:, None]


_tc_prep = pl.pallas_call(
    _tc_prep_body,
    grid=(GRID,),
    in_specs=[
        pl.BlockSpec((1, 1, ROW_BLK), lambda i: (i, 0, 0)),
        pl.BlockSpec((1, 1, ROW_BLK), lambda i: (i, 0, 0)),
        pl.BlockSpec((ROW_BLK, F), lambda i: (i, 0)),
        pl.BlockSpec((F, F), lambda i: (0, 0)),
    ],
    out_specs=[
        pl.BlockSpec((ROW_BLK, 1), lambda i: (i, 0)),
        pl.BlockSpec((ROW_BLK, F), lambda i: (i, 0)),
    ],
    out_shape=[
        jax.ShapeDtypeStruct((N, 1), jnp.float32),
        jax.ShapeDtypeStruct((N, F), jnp.float32),
    ],
)


def _make_tc_mid(kin, kout):
    def body(*refs):
        s = refs[:kin]
        hs = refs[kin:2 * kin]
        dinv_ref, b_ref, w_ref = refs[2 * kin], refs[2 * kin + 1], refs[2 * kin + 2]
        outs = refs[2 * kin + 3:]
        dinv = dinv_ref[...]
        x = jnp.concatenate(
            [s[k][0] + s[k][1] + hs[k][...] for k in range(kin)], axis=1
        )
        x = jax.nn.relu(x * dinv + b_ref[...])
        h = jnp.dot(x, w_ref[...], preferred_element_type=jnp.float32)
        for k in range(kout):
            outs[k][...] = h[:, k * F:(k + 1) * F] * dinv

    return pl.pallas_call(
        body,
        grid=(GRID,),
        in_specs=[pl.BlockSpec((NC, ROW_BLK, F), lambda i: (0, i, 0))] * kin
        + [pl.BlockSpec((ROW_BLK, F), lambda i: (i, 0))] * kin
        + [
            pl.BlockSpec((ROW_BLK, 1), lambda i: (i, 0)),
            pl.BlockSpec((1, kin * F), lambda i: (0, 0)),
            pl.BlockSpec((kin * F, kout * F), lambda i: (0, 0)),
        ],
        out_specs=[pl.BlockSpec((ROW_BLK, F), lambda i: (i, 0))] * kout,
        out_shape=[jax.ShapeDtypeStruct((N, F), jnp.float32)] * kout,
    )


_tc_mid2 = _make_tc_mid(1, 2)
_tc_mid3 = _make_tc_mid(2, 4)


def _tc_final_body(s0, s1, s2, s3, h0, h1, h2, h3r, dinv_ref, b3_ref, feat_ref,
                   pb_ref, fw1_ref, fb1_ref, fw2_ref, fb2_ref, out_ref, acc):
    i = pl.program_id(0)

    @pl.when(i == 0)
    def _():
        acc[...] = jnp.zeros_like(acc)

    dinv = dinv_ref[...]
    x = jnp.concatenate(
        [sr[0] + sr[1] + hr[...]
         for sr, hr in ((s0, h0), (s1, h1), (s2, h2), (s3, h3r))], axis=1
    )
    h3 = jax.nn.relu(x * dinv + b3_ref[...])
    gx = jnp.concatenate(
        [h3, feat_ref[...], jnp.ones((ROW_BLK, 1), jnp.float32)], axis=1
    )
    pb = pb_ref[0, 0, :]
    onehot = (
        lax.broadcasted_iota(jnp.int32, (G, ROW_BLK), 0) == pb[None, :]
    ).astype(jnp.float32)
    acc[...] += jnp.dot(onehot, gx, preferred_element_type=jnp.float32)

    @pl.when(i == GRID - 1)
    def _():
        sums = acc[:, : 5 * F]
        cnt = acc[:, 5 * F:]
        gc = sums / jnp.clip(cnt, 1.0)
        z = jax.nn.relu(
            jnp.dot(gc, fw1_ref[...], preferred_element_type=jnp.float32)
            + fb1_ref[...]
        )
        out_ref[...] = (
            jnp.dot(z, fw2_ref[...], preferred_element_type=jnp.float32)
            + fb2_ref[...]
        )


_tc_final = pl.pallas_call(
    _tc_final_body,
    grid=(GRID,),
    in_specs=[pl.BlockSpec((NC, ROW_BLK, F), lambda i: (0, i, 0))] * 4
    + [pl.BlockSpec((ROW_BLK, F), lambda i: (i, 0))] * 4
    + [
        pl.BlockSpec((ROW_BLK, 1), lambda i: (i, 0)),
        pl.BlockSpec((1, 4 * F), lambda i: (0, 0)),
        pl.BlockSpec((ROW_BLK, F), lambda i: (i, 0)),
        pl.BlockSpec((1, 1, ROW_BLK), lambda i: (i, 0, 0)),
        pl.BlockSpec((5 * F, 512), lambda i: (0, 0)),
        pl.BlockSpec((1, 512), lambda i: (0, 0)),
        pl.BlockSpec((512, 1), lambda i: (0, 0)),
        pl.BlockSpec((1, 1), lambda i: (0, 0)),
    ],
    out_specs=pl.BlockSpec((G, 1), lambda i: (0, 0)),
    out_shape=jax.ShapeDtypeStruct((G, 1), jnp.float32),
    scratch_shapes=[pltpu.VMEM((G, 5 * F + 1), jnp.float32)],
)


@jax.jit
def kernel(feature, edge_index, weight, protein_batch,
           W1, b1, W2, b2, W3, b3, fcW1, fcb1, fcW2, fcb2):
    src3 = edge_index[0].reshape(NC * NS, NB, B)
    dst3 = edge_index[1].reshape(NC * NS, NB, B)
    w3e = weight.reshape(NC * NS, NB, B)
    zeros = jnp.zeros((NS, RPT, F), jnp.float32)
    zdeg = jnp.zeros((NS, RPD), jnp.float32)
    pb3 = protein_batch.reshape(GRID, 1, ROW_BLK)

    degp = _deg_kernel(dst3, w3e, zdeg)
    d0 = degp[0].reshape(NP)[:N].reshape(GRID, 1, ROW_BLK)
    d1 = degp[1].reshape(NP)[:N].reshape(GRID, 1, ROW_BLK)
    dinv, hs1 = _tc_prep(d0, d1, feature, W1)
    (s1,) = _spmm1(src3, dst3, w3e, hs1, zeros)
    s1 = s1.reshape(NC, N, F)
    hs2 = _tc_mid2(s1, hs1, dinv, b1.reshape(1, F), W2)
    s2 = [o.reshape(NC, N, F) for o in _spmm2(src3, dst3, w3e, *hs2, zeros)]
    hs3 = _tc_mid3(*s2, *hs2, dinv, b2.reshape(1, 2 * F), W3)
    s3 = [o.reshape(NC, N, F) for o in _spmm3(src3, dst3, w3e, *hs3, zeros)]
    out = _tc_final(*s3, *hs3, dinv, b3.reshape(1, 4 * F), feature, pb3,
                    fcW1, fcb1.reshape(1, 512), fcW2, fcb2.reshape(1, 1))
    return out


# speedup vs baseline: 16.0656x; 1.1246x over previous
import functools

import jax
import jax.numpy as jnp
from jax import lax
from jax.experimental import pallas as pl
from jax.experimental.pallas import tpu as pltpu
from jax.experimental.pallas import tpu_sc as plsc

N = 10000
E = 320000
F = 128
G = 64
NC = 2
NS = 16
B = 80
EPT = E // (NC * NS)
NB = EPT // B
RPT = N // NS
ROW_BLK = 2000
GRID = N // ROW_BLK
NP = 10240
RPD = NP // NS


def _mesh():
    return plsc.VectorSubcoreMesh(
        core_axis_name="c", subcore_axis_name="s", num_cores=NC, num_subcores=NS
    )


def _deg_body(dst3, w3, zdeg, degp, dstv, wv, acc):
    c = lax.axis_index("c")
    s = lax.axis_index("s")
    wid = c * NS + s
    rlo = s * RPD
    pltpu.sync_copy(dst3.at[wid], dstv)
    pltpu.sync_copy(w3.at[wid], wv)
    pltpu.sync_copy(zdeg.at[s], acc.at[pl.ds(rlo, RPD)])
    plsc.subcore_barrier()

    def batch(i, _):
        pltpu.sync_copy(wv.at[i], acc.at[dstv.at[i]], add=True)
        return 0

    lax.fori_loop(0, NB, batch, 0)
    plsc.subcore_barrier()
    pltpu.sync_copy(acc.at[pl.ds(rlo, RPD)], degp.at[c, s])


_deg_kernel = pl.kernel(
    _deg_body,
    out_type=jax.ShapeDtypeStruct((NC, NS, RPD), jnp.float32),
    mesh=_mesh(),
    scratch_types=[
        pltpu.VMEM((NB, B), jnp.int32),
        pltpu.VMEM((NB, B), jnp.float32),
        pltpu.VMEM_SHARED((NP,), jnp.float32),
    ],
)


NSETS = 4


def _make_spmm(nchunk):
    ni = 4 + nchunk

    def body(*refs):
        src3, dst3, w3 = refs[0], refs[1], refs[2]
        hs = refs[3:3 + nchunk]
        zeros = refs[3 + nchunk]
        outs = refs[ni:ni + nchunk]
        sc = refs[ni + nchunk:]
        srcb, dstb, wb, rows, acc = sc[0], sc[1], sc[2], sc[3], sc[4]
        isem = sc[5:5 + NSETS]
        gsem = sc[5 + NSETS:5 + 2 * NSETS]
        ssem = sc[5 + 2 * NSETS:5 + 3 * NSETS]

        c = lax.axis_index("c")
        s = lax.axis_index("s")
        wid = c * NS + s

        def idx_start(i, t):
            pltpu.async_copy(src3.at[wid, i], srcb.at[t], isem[t])
            pltpu.async_copy(dst3.at[wid, i], dstb.at[t], isem[t])
            pltpu.async_copy(w3.at[wid, i], wb.at[t], isem[t])

        def idx_wait(i, t):
            pltpu.make_async_copy(src3.at[wid, i], srcb.at[t], isem[t]).wait()
            pltpu.make_async_copy(dst3.at[wid, i], dstb.at[t], isem[t]).wait()
            pltpu.make_async_copy(w3.at[wid, i], wb.at[t], isem[t]).wait()

        def gather_start(k, t):
            pltpu.async_copy(hs[k].at[srcb.at[t]], rows.at[t], gsem[t])

        def gather_wait(k, t):
            pltpu.make_async_copy(hs[k].at[srcb.at[t]], rows.at[t],
                                  gsem[t]).wait()

        def scat_start(t):
            pltpu.async_copy(rows.at[t], acc.at[dstb.at[t]], ssem[t], add=True)

        def scat_wait(t):
            pltpu.make_async_copy(rows.at[t], acc.at[dstb.at[t]],
                                  ssem[t]).wait()

        def mul(t):
            def mjg(jg, _):
                wchunk = wb[t, pl.ds(jg * 16, 16)]
                for r2 in range(16):
                    wsp = wchunk.at[jnp.full((16,), r2, jnp.int32)].get(
                        mode="promise_in_bounds")
                    r = jg * 16 + r2
                    for j in range(F // 16):
                        sl = pl.ds(j * 16, 16)
                        rows[t, r, sl] = rows[t, r, sl] * wsp
                return 0

            lax.fori_loop(0, B // 16, mjg, 0)

        NGRP = (NB + NSETS - 1) // NSETS

        rlo = s * RPT

        def prime(k):
            for t in range(NSETS):
                idx_start(t, t)
            for t in range(NSETS):
                idx_wait(t, t)
                gather_start(k, t)

        for k in range(nchunk):
            if k == 0:
                for t in range(NSETS):
                    idx_start(t, t)
                pltpu.sync_copy(zeros.at[s], acc.at[pl.ds(rlo, RPT)])
                for t in range(NSETS):
                    idx_wait(t, t)
                    gather_start(k, t)
                plsc.subcore_barrier()

            def step(g, t):
                i = NSETS * g + t
                tp = (t + NSETS - 1) % NSETS
                live = i < NB
                pref = (i >= 1) & (i + NSETS - 1 < NB)

                @pl.when(live)
                def _():
                    gather_wait(k, t)

                @pl.when(pref)
                def _():
                    scat_wait(tp)
                    idx_start(i + NSETS - 1, tp)

                @pl.when(live)
                def _():
                    mul(t)
                    scat_start(t)

                @pl.when(pref)
                def _():
                    idx_wait(i + NSETS - 1, tp)
                    gather_start(k, tp)

            def group(g, _):
                for t in range(NSETS):
                    step(g, t)
                return 0

            lax.fori_loop(0, NGRP, group, 0)

            for j in range(NSETS):
                scat_wait((NB - NSETS + j) % NSETS)

            if k + 1 < nchunk:
                prime(k + 1)
            plsc.subcore_barrier()
            pltpu.sync_copy(acc.at[pl.ds(rlo, RPT)], outs[k].at[c, s])
            if k + 1 < nchunk:
                pltpu.sync_copy(zeros.at[s], acc.at[pl.ds(rlo, RPT)])
                plsc.subcore_barrier()

    return pl.kernel(
        body,
        out_type=[jax.ShapeDtypeStruct((NC, NS, RPT, F), jnp.float32)] * nchunk,
        mesh=_mesh(),
        scratch_types=[
            pltpu.VMEM((NSETS, B), jnp.int32),
            pltpu.VMEM((NSETS, B), jnp.int32),
            pltpu.VMEM((NSETS, B), jnp.float32),
            pltpu.VMEM((NSETS, B, F), jnp.float32),
            pltpu.VMEM_SHARED((N, F), jnp.float32),
        ] + [pltpu.SemaphoreType.DMA] * (3 * NSETS),
    )


_spmm1 = _make_spmm(1)
_spmm2 = _make_spmm(2)
_spmm3 = _make_spmm(4)


def _tc_prep_body(d0_ref, d1_ref, feat_ref, w1_ref, dinv_ref, hs1_ref):
    deg = d0_ref[0, 0, :] + d1_ref[0, 0, :] + 1.0
    dinv = jnp.where(deg > 0, lax.rsqrt(deg), 0.0)
    h = jnp.dot(feat_ref[...], w1_ref[...], preferred_element_type=jnp.float32)
    hs1_ref[...] = h * dinv[:, None]
    dinv_ref[...] = dinv[:, None]


_tc_prep = pl.pallas_call(
    _tc_prep_body,
    grid=(GRID,),
    in_specs=[
        pl.BlockSpec((1, 1, ROW_BLK), lambda i: (i, 0, 0)),
        pl.BlockSpec((1, 1, ROW_BLK), lambda i: (i, 0, 0)),
        pl.BlockSpec((ROW_BLK, F), lambda i: (i, 0)),
        pl.BlockSpec((F, F), lambda i: (0, 0)),
    ],
    out_specs=[
        pl.BlockSpec((ROW_BLK, 1), lambda i: (i, 0)),
        pl.BlockSpec((ROW_BLK, F), lambda i: (i, 0)),
    ],
    out_shape=[
        jax.ShapeDtypeStruct((N, 1), jnp.float32),
        jax.ShapeDtypeStruct((N, F), jnp.float32),
    ],
)


def _make_tc_mid(kin, kout):
    def body(*refs):
        s = refs[:kin]
        hs = refs[kin:2 * kin]
        dinv_ref, b_ref, w_ref = refs[2 * kin], refs[2 * kin + 1], refs[2 * kin + 2]
        outs = refs[2 * kin + 3:]
        dinv = dinv_ref[...]
        x = jnp.concatenate(
            [s[k][0] + s[k][1] + hs[k][...] for k in range(kin)], axis=1
        )
        x = jax.nn.relu(x * dinv + b_ref[...])
        h = jnp.dot(x, w_ref[...], preferred_element_type=jnp.float32)
        for k in range(kout):
            outs[k][...] = h[:, k * F:(k + 1) * F] * dinv

    return pl.pallas_call(
        body,
        grid=(GRID,),
        in_specs=[pl.BlockSpec((NC, ROW_BLK, F), lambda i: (0, i, 0))] * kin
        + [pl.BlockSpec((ROW_BLK, F), lambda i: (i, 0))] * kin
        + [
            pl.BlockSpec((ROW_BLK, 1), lambda i: (i, 0)),
            pl.BlockSpec((1, kin * F), lambda i: (0, 0)),
            pl.BlockSpec((kin * F, kout * F), lambda i: (0, 0)),
        ],
        out_specs=[pl.BlockSpec((ROW_BLK, F), lambda i: (i, 0))] * kout,
        out_shape=[jax.ShapeDtypeStruct((N, F), jnp.float32)] * kout,
    )


_tc_mid2 = _make_tc_mid(1, 2)
_tc_mid3 = _make_tc_mid(2, 4)


def _tc_final_body(s0, s1, s2, s3, h0, h1, h2, h3r, dinv_ref, b3_ref, feat_ref,
                   pb_ref, fw1_ref, fb1_ref, fw2_ref, fb2_ref, out_ref, acc):
    i = pl.program_id(0)

    @pl.when(i == 0)
    def _():
        acc[...] = jnp.zeros_like(acc)

    dinv = dinv_ref[...]
    x = jnp.concatenate(
        [sr[0] + sr[1] + hr[...]
         for sr, hr in ((s0, h0), (s1, h1), (s2, h2), (s3, h3r))], axis=1
    )
    h3 = jax.nn.relu(x * dinv + b3_ref[...])
    gx = jnp.concatenate(
        [h3, feat_ref[...], jnp.ones((ROW_BLK, 1), jnp.float32)], axis=1
    )
    pb = pb_ref[0, 0, :]
    onehot = (
        lax.broadcasted_iota(jnp.int32, (G, ROW_BLK), 0) == pb[None, :]
    ).astype(jnp.float32)
    acc[...] += jnp.dot(onehot, gx, preferred_element_type=jnp.float32)

    @pl.when(i == GRID - 1)
    def _():
        sums = acc[:, : 5 * F]
        cnt = acc[:, 5 * F:]
        gc = sums / jnp.clip(cnt, 1.0)
        z = jax.nn.relu(
            jnp.dot(gc, fw1_ref[...], preferred_element_type=jnp.float32)
            + fb1_ref[...]
        )
        out_ref[...] = (
            jnp.dot(z, fw2_ref[...], preferred_element_type=jnp.float32)
            + fb2_ref[...]
        )


_tc_final = pl.pallas_call(
    _tc_final_body,
    grid=(GRID,),
    in_specs=[pl.BlockSpec((NC, ROW_BLK, F), lambda i: (0, i, 0))] * 4
    + [pl.BlockSpec((ROW_BLK, F), lambda i: (i, 0))] * 4
    + [
        pl.BlockSpec((ROW_BLK, 1), lambda i: (i, 0)),
        pl.BlockSpec((1, 4 * F), lambda i: (0, 0)),
        pl.BlockSpec((ROW_BLK, F), lambda i: (i, 0)),
        pl.BlockSpec((1, 1, ROW_BLK), lambda i: (i, 0, 0)),
        pl.BlockSpec((5 * F, 512), lambda i: (0, 0)),
        pl.BlockSpec((1, 512), lambda i: (0, 0)),
        pl.BlockSpec((512, 1), lambda i: (0, 0)),
        pl.BlockSpec((1, 1), lambda i: (0, 0)),
    ],
    out_specs=pl.BlockSpec((G, 1), lambda i: (0, 0)),
    out_shape=jax.ShapeDtypeStruct((G, 1), jnp.float32),
    scratch_shapes=[pltpu.VMEM((G, 5 * F + 1), jnp.float32)],
)


@jax.jit
def kernel(feature, edge_index, weight, protein_batch,
           W1, b1, W2, b2, W3, b3, fcW1, fcb1, fcW2, fcb2):
    src3 = edge_index[0].reshape(NC * NS, NB, B)
    dst3 = edge_index[1].reshape(NC * NS, NB, B)
    w3e = weight.reshape(NC * NS, NB, B)
    zeros = jnp.zeros((NS, RPT, F), jnp.float32)
    zdeg = jnp.zeros((NS, RPD), jnp.float32)
    pb3 = protein_batch.reshape(GRID, 1, ROW_BLK)

    degp = _deg_kernel(dst3, w3e, zdeg)
    d0 = degp[0].reshape(NP)[:N].reshape(GRID, 1, ROW_BLK)
    d1 = degp[1].reshape(NP)[:N].reshape(GRID, 1, ROW_BLK)
    dinv, hs1 = _tc_prep(d0, d1, feature, W1)
    (s1,) = _spmm1(src3, dst3, w3e, hs1, zeros)
    s1 = s1.reshape(NC, N, F)
    hs2 = _tc_mid2(s1, hs1, dinv, b1.reshape(1, F), W2)
    s2 = [o.reshape(NC, N, F) for o in _spmm2(src3, dst3, w3e, *hs2, zeros)]
    hs3 = _tc_mid3(*s2, *hs2, dinv, b2.reshape(1, 2 * F), W3)
    s3 = [o.reshape(NC, N, F) for o in _spmm3(src3, dst3, w3e, *hs3, zeros)]
    out = _tc_final(*s3, *hs3, dinv, b3.reshape(1, 4 * F), feature, pb3,
                    fcW1, fcb1.reshape(1, 512), fcW2, fcb2.reshape(1, 1))
    return out
